# Initial kernel scaffold; baseline (speedup 1.0000x reference)
import functools

import jax
import jax.numpy as jnp
from jax import lax
from jax.experimental import pallas as pl
from jax.experimental.pallas import tpu as pltpu
from jax.experimental.pallas import tpu_sc as plsc

_VOCAB = 100000
_DIM = 64
_BATCH = 4096
_HIST = 50

_N = _BATCH * _HIST
_NUM_WORKERS = 32
_PER_W = _N // _NUM_WORKERS
_CHUNK = 128
_NCHUNK = _PER_W // _CHUNK
_NBUF = 2


def _emb_body(idx_hbm, table_hbm, out_hbm, idx_v, rows_v, sems):
    wid = lax.axis_index("s") * 2 + lax.axis_index("c")
    base = wid * _PER_W

    pltpu.sync_copy(idx_hbm.at[pl.ds(wid * _NCHUNK, _NCHUNK)], idx_v)

    def gather(c, buf):
        return pltpu.async_copy(
            table_hbm.at[idx_v.at[c]], rows_v.at[buf], sems.at[buf]
        )

    gather(0, 0)

    def body(c, _):
        buf = lax.rem(c, _NBUF)
        nxt = lax.rem(c + 1, _NBUF)

        @pl.when(c + 1 < _NCHUNK)
        def _():
            gather(c + 1, nxt)

        pltpu.make_async_copy(
            table_hbm.at[idx_v.at[c]], rows_v.at[buf], sems.at[buf]
        ).wait()
        pltpu.sync_copy(rows_v.at[buf], out_hbm.at[pl.ds(base + c * _CHUNK, _CHUNK)])
        return _

    lax.fori_loop(0, _NCHUNK, body, 0)


_emb_kernel = functools.partial(
    pl.kernel,
    out_type=jax.ShapeDtypeStruct((_N, _DIM), jnp.float32),
    mesh=plsc.VectorSubcoreMesh(core_axis_name="c", subcore_axis_name="s"),
    scratch_types=[
        pltpu.VMEM((_NCHUNK, _CHUNK), jnp.int32),
        pltpu.VMEM((_NBUF, _CHUNK, _DIM), jnp.float32),
        pltpu.SemaphoreType.DMA((_NBUF,)),
    ],
)(_emb_body)


@jax.jit
def kernel(indices, embedding):
    idx2d = indices.reshape(_N // _CHUNK, _CHUNK)
    out = _emb_kernel(idx2d, embedding)
    return out.reshape(_BATCH, _HIST, _DIM)


# speedup vs baseline: 4.5330x; 4.5330x over previous
import functools

import jax
import jax.numpy as jnp
from jax import lax
from jax.experimental import pallas as pl
from jax.experimental.pallas import tpu as pltpu
from jax.experimental.pallas import tpu_sc as plsc

_VOCAB = 100000
_DIM = 64
_BATCH = 4096
_HIST = 50

_N = _BATCH * _HIST
_NUM_WORKERS = 32
_PER_W = _N // _NUM_WORKERS
_CHUNK = 128
_NCHUNK = _PER_W // _CHUNK
_NBUF = 2


def _emb_body(idx_hbm, table_hbm, out_hbm, idx_v, rows_v, sems):
    wid = lax.axis_index("s") * 2 + lax.axis_index("c")
    base = wid * _PER_W

    pltpu.sync_copy(idx_hbm.at[wid], idx_v)

    def gather(c, buf):
        return pltpu.async_copy(
            table_hbm.at[idx_v.at[c]], rows_v.at[buf], sems.at[buf]
        )

    gather(0, 0)

    def body(c, carry):
        buf = lax.rem(c, _NBUF)
        nxt = lax.rem(c + 1, _NBUF)

        @pl.when(c + 1 < _NCHUNK)
        def _prefetch():
            gather(c + 1, nxt)

        pltpu.make_async_copy(
            table_hbm.at[idx_v.at[c]], rows_v.at[buf], sems.at[buf]
        ).wait()
        pltpu.sync_copy(rows_v.at[buf], out_hbm.at[pl.ds(base + c * _CHUNK, _CHUNK)])
        return carry

    lax.fori_loop(0, _NCHUNK, body, 0)


_emb_kernel = functools.partial(
    pl.kernel,
    out_type=jax.ShapeDtypeStruct((_N, _DIM), jnp.float32),
    mesh=plsc.VectorSubcoreMesh(core_axis_name="c", subcore_axis_name="s"),
    scratch_types=[
        pltpu.VMEM((_NCHUNK, _CHUNK), jnp.int32),
        pltpu.VMEM((_NBUF, _CHUNK, _DIM), jnp.float32),
        pltpu.SemaphoreType.DMA((_NBUF,)),
    ],
    compiler_params=pltpu.CompilerParams(use_tc_tiling_on_sc=False),
)(_emb_body)


@jax.jit
def kernel(indices, embedding):
    idx3d = indices.reshape(_NUM_WORKERS, _NCHUNK, _CHUNK)
    out = _emb_kernel(idx3d, embedding)
    return out.reshape(_BATCH, _HIST, _DIM)


# speedup vs baseline: 4.6689x; 1.0300x over previous
import functools

import jax
import jax.numpy as jnp
from jax import lax
from jax.experimental import pallas as pl
from jax.experimental.pallas import tpu as pltpu
from jax.experimental.pallas import tpu_sc as plsc

_VOCAB = 100000
_DIM = 64
_BATCH = 4096
_HIST = 50

_N = _BATCH * _HIST
_NUM_WORKERS = 32
_PER_W = _N // _NUM_WORKERS
_CHUNK = 128
_NCHUNK = _PER_W // _CHUNK
_NBUF = 8
_LEAD = 4


def _emb_body(idx_hbm, table_hbm, out_hbm, idx_v, rows_v, gsems, osems):
    wid = lax.axis_index("s") * 2 + lax.axis_index("c")
    base = wid * _PER_W

    pltpu.sync_copy(idx_hbm.at[wid], idx_v)

    def gather(c, buf):
        pltpu.async_copy(table_hbm.at[idx_v.at[c]], rows_v.at[buf], gsems.at[buf])

    def wait_gather(buf):
        pltpu.make_async_copy(
            table_hbm.at[idx_v.at[0]], rows_v.at[buf], gsems.at[buf]
        ).wait()

    def put(c, buf):
        pltpu.async_copy(
            rows_v.at[buf], out_hbm.at[pl.ds(base + c * _CHUNK, _CHUNK)], osems.at[buf]
        )

    def wait_put(buf):
        pltpu.make_async_copy(
            rows_v.at[buf], out_hbm.at[pl.ds(base, _CHUNK)], osems.at[buf]
        ).wait()

    for b in range(_LEAD):
        gather(b, b)

    def body(c, carry):
        buf = lax.rem(c, _NBUF)
        wait_gather(buf)
        put(c, buf)

        nc = c + _LEAD
        nbuf = lax.rem(nc, _NBUF)

        @pl.when(jnp.logical_and(nc < _NCHUNK, nc >= _NBUF))
        def _recycle():
            wait_put(nbuf)

        @pl.when(nc < _NCHUNK)
        def _prefetch():
            gather(nc, nbuf)

        return carry

    lax.fori_loop(0, _NCHUNK, body, 0)

    for b in range(_NBUF):
        wait_put(b)


_emb_kernel = functools.partial(
    pl.kernel,
    out_type=jax.ShapeDtypeStruct((_N, _DIM), jnp.float32),
    mesh=plsc.VectorSubcoreMesh(core_axis_name="c", subcore_axis_name="s"),
    scratch_types=[
        pltpu.VMEM((_NCHUNK, _CHUNK), jnp.int32),
        pltpu.VMEM((_NBUF, _CHUNK, _DIM), jnp.float32),
        pltpu.SemaphoreType.DMA((_NBUF,)),
        pltpu.SemaphoreType.DMA((_NBUF,)),
    ],
    compiler_params=pltpu.CompilerParams(use_tc_tiling_on_sc=False),
)(_emb_body)


@jax.jit
def kernel(indices, embedding):
    idx3d = indices.reshape(_NUM_WORKERS, _NCHUNK, _CHUNK)
    out = _emb_kernel(idx3d, embedding)
    return out.reshape(_BATCH, _HIST, _DIM)
